# baseline (speedup 1.0000x reference)
import functools
import math

import jax
import jax.numpy as jnp
from jax.experimental import pallas as pl
from jax.experimental.pallas import tpu as pltpu

_TB = 64
_NCAND = 6


def _sorted_desc_extract(arr, n, neg_inf):
    m = jnp.max(arr, axis=1, keepdims=True)
    vals = [m]
    for _ in range(1, n):
        cand = jnp.where(arr < m, arr, neg_inf)
        m = jnp.max(cand, axis=1, keepdims=True)
        vals.append(m)
    return jnp.concatenate(vals, axis=1)


def _body(q_ref, k_ref, v_ref, sal_ref, rv_ref, aw_ref, sa, sb, vals_ref,
          *, n_top, inv_sqrt_d):
    i = pl.program_id(0)

    def produce(dst):
        s = jax.lax.dot_general(
            q_ref[...], k_ref[...], (((1,), (1,)), ((), ())),
            preferred_element_type=jnp.float32)
        dst[...] = s * inv_sqrt_d + sal_ref[...]

    def consume(src):
        scores = src[...]
        tb, s_dim = scores.shape
        neg_inf = jnp.float32(-jnp.inf)

        s3 = scores.reshape(tb, s_dim // 128, 128)
        m = jnp.max(s3, axis=1)
        cands = [m]
        for _ in range(1, _NCAND):
            masked = jnp.where(s3 < m[:, None, :], s3, neg_inf)
            m = jnp.max(masked, axis=1)
            cands.append(m)
        u = jnp.concatenate(cands, axis=1)

        vals_ref[...] = _sorted_desc_extract(u, n_top, neg_inf)

        th_hat = vals_ref[:, n_top - 1:n_top]
        cnt = jnp.sum((scores > th_hat).astype(jnp.float32), axis=1,
                      keepdims=True)
        bad = jnp.sum(jnp.where(cnt > (n_top - 0.5),
                                jnp.float32(1.0), jnp.float32(0.0)))

        @pl.when(bad > 0)
        def _():
            vals_ref[...] = _sorted_desc_extract(scores, n_top, neg_inf)

        vals = vals_ref[...]
        m0 = vals[:, 0:1]
        e = jnp.exp(vals - m0)
        denom = jnp.sum(e, axis=1, keepdims=True)
        aw_ref[...] = e / denom

        thresh = vals[:, n_top - 1:n_top]
        z = m0 + jnp.log(denom)
        wd = jnp.where(scores >= thresh,
                       jnp.exp(scores - z),
                       jnp.float32(0.0)).astype(jnp.bfloat16)
        rv_ref[...] = jax.lax.dot_general(
            wd, v_ref[...], (((1,), (0,)), ((), ())),
            preferred_element_type=jnp.float32)

    @pl.when(i % 2 == 0)
    def _():
        consume(sb)
        produce(sa)

    @pl.when(i % 2 == 1)
    def _():
        consume(sa)
        produce(sb)


def kernel(q, K, V, salience, topk):
    Bq, Tq, Dq = q.shape
    S = K.shape[0]
    n_top = min(32, S)
    R = Bq * Tq
    N = R // _TB
    q2 = q.reshape(R, Dq)
    sal2 = salience.reshape(1, S)
    v16 = V.astype(jnp.bfloat16)

    body = functools.partial(_body, n_top=n_top,
                             inv_sqrt_d=float(1.0 / math.sqrt(Dq)))
    rv, aw = pl.pallas_call(
        body,
        grid=(N + 1,),
        in_specs=[
            pl.BlockSpec((_TB, Dq), lambda i: (jnp.minimum(i, N - 1), 0)),
            pl.BlockSpec((S, Dq), lambda i: (0, 0)),
            pl.BlockSpec((S, Dq), lambda i: (0, 0)),
            pl.BlockSpec((1, S), lambda i: (0, 0)),
        ],
        out_specs=[
            pl.BlockSpec((_TB, Dq), lambda i: (jnp.maximum(i - 1, 0), 0)),
            pl.BlockSpec((_TB, n_top), lambda i: (jnp.maximum(i - 1, 0), 0)),
        ],
        out_shape=[
            jax.ShapeDtypeStruct((R, Dq), jnp.float32),
            jax.ShapeDtypeStruct((R, n_top), jnp.float32),
        ],
        scratch_shapes=[
            pltpu.VMEM((_TB, S), jnp.float32),
            pltpu.VMEM((_TB, S), jnp.float32),
            pltpu.VMEM((_TB, n_top), jnp.float32),
        ],
        compiler_params=pltpu.CompilerParams(
            dimension_semantics=("arbitrary",),
        ),
    )(q2, K, v16, sal2)
    return rv.reshape(Bq, Tq, Dq), aw.reshape(Bq, Tq, n_top)


# speedup vs baseline: 1.8102x; 1.8102x over previous
import functools
import math

import jax
import jax.numpy as jnp
from jax.experimental import pallas as pl
from jax.experimental.pallas import tpu as pltpu

_TB = 128
_NCAND = 6


def _sorted_desc_extract(arr, n, neg_inf):
    m = jnp.max(arr, axis=1, keepdims=True)
    vals = [m]
    for _ in range(1, n):
        cand = jnp.where(arr < m, arr, neg_inf)
        m = jnp.max(cand, axis=1, keepdims=True)
        vals.append(m)
    return jnp.concatenate(vals, axis=1)


def _body(q_ref, k_ref, v_ref, sal_ref, rv_ref, aw_ref, vals_ref,
          *, n_top, inv_sqrt_d):
    scores = jax.lax.dot_general(
        q_ref[...], k_ref[...], (((1,), (1,)), ((), ())),
        preferred_element_type=jnp.float32)
    scores = scores * inv_sqrt_d + sal_ref[...]
    tb, s_dim = scores.shape
    neg_inf = jnp.float32(-jnp.inf)

    s3 = scores.reshape(tb, s_dim // 128, 128)
    m = jnp.max(s3, axis=1)
    cands = [m]
    for _ in range(1, _NCAND):
        masked = jnp.where(s3 < m[:, None, :], s3, neg_inf)
        m = jnp.max(masked, axis=1)
        cands.append(m)
    u = jnp.concatenate(cands, axis=1)

    vals_ref[...] = _sorted_desc_extract(u, n_top, neg_inf)

    th_hat = vals_ref[:, n_top - 1:n_top]
    cnt = jnp.sum((scores > th_hat).astype(jnp.float32), axis=1,
                  keepdims=True)
    bad = jnp.sum(jnp.where(cnt > (n_top - 0.5),
                            jnp.float32(1.0), jnp.float32(0.0)))

    @pl.when(bad > 0)
    def _():
        vals_ref[...] = _sorted_desc_extract(scores, n_top, neg_inf)

    vals = vals_ref[...]
    m0 = vals[:, 0:1]
    e = jnp.exp(vals - m0)
    denom = jnp.sum(e, axis=1, keepdims=True)
    aw_ref[...] = e / denom

    thresh = vals[:, n_top - 1:n_top]
    z = m0 + jnp.log(denom)
    wd = jnp.where(scores >= thresh,
                   jnp.exp(scores - z),
                   jnp.float32(0.0)).astype(jnp.bfloat16)
    rv_ref[...] = jax.lax.dot_general(
        wd, v_ref[...], (((1,), (0,)), ((), ())),
        preferred_element_type=jnp.float32)


def kernel(q, K, V, salience, topk):
    Bq, Tq, Dq = q.shape
    S = K.shape[0]
    n_top = min(32, S)
    R = Bq * Tq
    q2 = q.reshape(R, Dq)
    sal2 = salience.reshape(1, S)
    v16 = V.astype(jnp.bfloat16)

    body = functools.partial(_body, n_top=n_top,
                             inv_sqrt_d=float(1.0 / math.sqrt(Dq)))
    rv, aw = pl.pallas_call(
        body,
        grid=(R // _TB,),
        in_specs=[
            pl.BlockSpec((_TB, Dq), lambda i: (i, 0)),
            pl.BlockSpec((S, Dq), lambda i: (0, 0)),
            pl.BlockSpec((S, Dq), lambda i: (0, 0)),
            pl.BlockSpec((1, S), lambda i: (0, 0)),
        ],
        out_specs=[
            pl.BlockSpec((_TB, Dq), lambda i: (i, 0)),
            pl.BlockSpec((_TB, n_top), lambda i: (i, 0)),
        ],
        out_shape=[
            jax.ShapeDtypeStruct((R, Dq), jnp.float32),
            jax.ShapeDtypeStruct((R, n_top), jnp.float32),
        ],
        scratch_shapes=[
            pltpu.VMEM((_TB, n_top), jnp.float32),
        ],
        compiler_params=pltpu.CompilerParams(
            dimension_semantics=("arbitrary",),
        ),
    )(q2, K, v16, sal2)
    return rv.reshape(Bq, Tq, Dq), aw.reshape(Bq, Tq, n_top)


# speedup vs baseline: 1.8638x; 1.0296x over previous
import functools
import math

import jax
import jax.numpy as jnp
from jax.experimental import pallas as pl
from jax.experimental.pallas import tpu as pltpu

_TB = 128
_NCAND = 5


def _sorted_desc_extract(arr, n, neg_inf):
    m = jnp.max(arr, axis=1, keepdims=True)
    vals = [m]
    for _ in range(1, n):
        cand = jnp.where(arr < m, arr, neg_inf)
        m = jnp.max(cand, axis=1, keepdims=True)
        vals.append(m)
    return jnp.concatenate(vals, axis=1)


def _body(q_ref, k_ref, v_ref, sal_ref, rv_ref, aw_ref, vals_ref,
          *, n_top, inv_sqrt_d):
    scores = jax.lax.dot_general(
        q_ref[...], k_ref[...], (((1,), (1,)), ((), ())),
        preferred_element_type=jnp.float32)
    scores = scores * inv_sqrt_d + sal_ref[...]
    tb, s_dim = scores.shape
    neg_inf = jnp.float32(-jnp.inf)

    s3 = scores.reshape(tb, s_dim // 128, 128)
    m = jnp.max(s3, axis=1)
    cands = [m]
    for _ in range(1, _NCAND):
        masked = jnp.where(s3 < m[:, None, :], s3, neg_inf)
        m = jnp.max(masked, axis=1)
        cands.append(m)
    u = jnp.concatenate(cands, axis=1)

    vals_ref[...] = _sorted_desc_extract(u, n_top, neg_inf)

    th_hat = vals_ref[:, n_top - 1:n_top]
    cnt = jnp.sum((scores > th_hat).astype(jnp.float32), axis=1,
                  keepdims=True)
    bad = jnp.sum(jnp.where(cnt > (n_top - 0.5),
                            jnp.float32(1.0), jnp.float32(0.0)))

    @pl.when(bad > 0)
    def _():
        vals_ref[...] = _sorted_desc_extract(scores, n_top, neg_inf)

    vals = vals_ref[...]
    m0 = vals[:, 0:1]
    e = jnp.exp(vals - m0)
    denom = jnp.sum(e, axis=1, keepdims=True)
    aw_ref[...] = e / denom

    thresh = vals[:, n_top - 1:n_top]
    z = m0 + jnp.log(denom)
    wd = jnp.where(scores >= thresh,
                   jnp.exp(scores - z),
                   jnp.float32(0.0)).astype(jnp.bfloat16)
    rv_ref[...] = jax.lax.dot_general(
        wd, v_ref[...], (((1,), (0,)), ((), ())),
        preferred_element_type=jnp.float32)


def kernel(q, K, V, salience, topk):
    Bq, Tq, Dq = q.shape
    S = K.shape[0]
    n_top = min(32, S)
    R = Bq * Tq
    q2 = q.reshape(R, Dq)
    sal2 = salience.reshape(1, S)
    v16 = V.astype(jnp.bfloat16)

    body = functools.partial(_body, n_top=n_top,
                             inv_sqrt_d=float(1.0 / math.sqrt(Dq)))
    rv, aw = pl.pallas_call(
        body,
        grid=(R // _TB,),
        in_specs=[
            pl.BlockSpec((_TB, Dq), lambda i: (i, 0)),
            pl.BlockSpec((S, Dq), lambda i: (0, 0)),
            pl.BlockSpec((S, Dq), lambda i: (0, 0)),
            pl.BlockSpec((1, S), lambda i: (0, 0)),
        ],
        out_specs=[
            pl.BlockSpec((_TB, Dq), lambda i: (i, 0)),
            pl.BlockSpec((_TB, n_top), lambda i: (i, 0)),
        ],
        out_shape=[
            jax.ShapeDtypeStruct((R, Dq), jnp.float32),
            jax.ShapeDtypeStruct((R, n_top), jnp.float32),
        ],
        scratch_shapes=[
            pltpu.VMEM((_TB, n_top), jnp.float32),
        ],
        compiler_params=pltpu.CompilerParams(
            dimension_semantics=("arbitrary",),
        ),
    )(q2, K, v16, sal2)
    return rv.reshape(Bq, Tq, Dq), aw.reshape(Bq, Tq, n_top)


# speedup vs baseline: 1.8907x; 1.0144x over previous
import functools
import math

import jax
import jax.numpy as jnp
from jax.experimental import pallas as pl
from jax.experimental.pallas import tpu as pltpu

_TB = 128
_NCAND = 4


def _sorted_desc_extract(arr, n, neg_inf):
    m = jnp.max(arr, axis=1, keepdims=True)
    vals = [m]
    for _ in range(1, n):
        cand = jnp.where(arr < m, arr, neg_inf)
        m = jnp.max(cand, axis=1, keepdims=True)
        vals.append(m)
    return jnp.concatenate(vals, axis=1)


def _body(q_ref, k_ref, v_ref, sal_ref, rv_ref, aw_ref, vals_ref,
          *, n_top, inv_sqrt_d):
    scores = jax.lax.dot_general(
        q_ref[...], k_ref[...], (((1,), (1,)), ((), ())),
        preferred_element_type=jnp.float32)
    scores = scores * inv_sqrt_d + sal_ref[...]
    tb, s_dim = scores.shape
    neg_inf = jnp.float32(-jnp.inf)

    s3 = scores.reshape(tb, s_dim // 128, 128)
    m = jnp.max(s3, axis=1)
    cands = [m]
    for _ in range(1, _NCAND):
        masked = jnp.where(s3 < m[:, None, :], s3, neg_inf)
        m = jnp.max(masked, axis=1)
        cands.append(m)
    u = jnp.concatenate(cands, axis=1)

    vals_ref[...] = _sorted_desc_extract(u, n_top, neg_inf)

    th_hat = vals_ref[:, n_top - 1:n_top]
    cnt = jnp.sum((scores > th_hat).astype(jnp.float32), axis=1,
                  keepdims=True)
    bad = jnp.sum(jnp.where(cnt > (n_top - 0.5),
                            jnp.float32(1.0), jnp.float32(0.0)))

    @pl.when(bad > 0)
    def _():
        vals_ref[...] = _sorted_desc_extract(scores, n_top, neg_inf)

    vals = vals_ref[...]
    m0 = vals[:, 0:1]
    e = jnp.exp(vals - m0)
    denom = jnp.sum(e, axis=1, keepdims=True)
    aw_ref[...] = e / denom

    thresh = vals[:, n_top - 1:n_top]
    z = m0 + jnp.log(denom)
    wd = jnp.where(scores >= thresh,
                   jnp.exp(scores - z),
                   jnp.float32(0.0)).astype(jnp.bfloat16)
    rv_ref[...] = jax.lax.dot_general(
        wd, v_ref[...], (((1,), (0,)), ((), ())),
        preferred_element_type=jnp.float32)


def kernel(q, K, V, salience, topk):
    Bq, Tq, Dq = q.shape
    S = K.shape[0]
    n_top = min(32, S)
    R = Bq * Tq
    q2 = q.reshape(R, Dq)
    sal2 = salience.reshape(1, S)
    v16 = V.astype(jnp.bfloat16)

    body = functools.partial(_body, n_top=n_top,
                             inv_sqrt_d=float(1.0 / math.sqrt(Dq)))
    rv, aw = pl.pallas_call(
        body,
        grid=(R // _TB,),
        in_specs=[
            pl.BlockSpec((_TB, Dq), lambda i: (i, 0)),
            pl.BlockSpec((S, Dq), lambda i: (0, 0)),
            pl.BlockSpec((S, Dq), lambda i: (0, 0)),
            pl.BlockSpec((1, S), lambda i: (0, 0)),
        ],
        out_specs=[
            pl.BlockSpec((_TB, Dq), lambda i: (i, 0)),
            pl.BlockSpec((_TB, n_top), lambda i: (i, 0)),
        ],
        out_shape=[
            jax.ShapeDtypeStruct((R, Dq), jnp.float32),
            jax.ShapeDtypeStruct((R, n_top), jnp.float32),
        ],
        scratch_shapes=[
            pltpu.VMEM((_TB, n_top), jnp.float32),
        ],
        compiler_params=pltpu.CompilerParams(
            dimension_semantics=("arbitrary",),
        ),
    )(q2, K, v16, sal2)
    return rv.reshape(Bq, Tq, Dq), aw.reshape(Bq, Tq, n_top)
